# Initial kernel scaffold; baseline (speedup 1.0000x reference)
import jax
import jax.numpy as jnp
from jax.experimental import pallas as pl


def kernel(inputs, offsets, adj_matrix, word_vectors, W, a_src, a_dst):
    raise NotImplementedError("write your pallas kernel here")


# speedup vs baseline: 1.0506x; 1.0506x over previous
import functools

import jax
import jax.numpy as jnp
from jax.experimental import pallas as pl
from jax.experimental.pallas import tpu as pltpu

_INTERPRET = False


def _vocab_matmul_body(wv_ref, w_ref, out_ref):
    out_ref[...] = jnp.dot(wv_ref[...], w_ref[...],
                           preferred_element_type=jnp.float32,
                           precision=jax.lax.Precision.HIGHEST)


def _vocab_matmul(wv, W, bv):
    V, D = wv.shape
    out_dim = W.shape[1]
    return pl.pallas_call(
        _vocab_matmul_body,
        grid=(V // bv,),
        in_specs=[
            pl.BlockSpec((bv, D), lambda i: (i, 0)),
            pl.BlockSpec((D, out_dim), lambda i: (0, 0)),
        ],
        out_specs=pl.BlockSpec((bv, out_dim), lambda i: (i, 0)),
        out_shape=jax.ShapeDtypeStruct((V, out_dim), jnp.float32),
        interpret=_INTERPRET,
    )(wv, W)


def _flash_gat_body(adj_ref, sums_r_ref, sums_c_ref, ic_r_ref, ic_c_ref,
                    av_ref, out_ref):
    hr = sums_r_ref[...] * ic_r_ref[...]
    hc = sums_c_ref[...] * ic_c_ref[...]
    a00 = av_ref[0, 0]
    a01 = av_ref[0, 1]
    a10 = av_ref[0, 2]
    a11 = av_ref[0, 3]
    s = hr[:, 0] * a00 + hr[:, 1] * a01
    t = hc[0, :] * a10 + hc[1, :] * a11
    e = s[:, None] + t[None, :]
    e = jnp.where(e >= 0, e, 0.2 * e)
    e = jnp.where(adj_ref[...] > 0.5, e, jnp.float32(-1e9))
    m = jnp.max(e, axis=1, keepdims=True)
    p = jnp.exp(e - m)
    denom = jnp.sum(p, axis=1, keepdims=True)
    alpha = (p / denom).astype(jnp.bfloat16).astype(jnp.float32)
    hx = hc[0, :].astype(jnp.bfloat16).astype(jnp.float32)
    hy = hc[1, :].astype(jnp.bfloat16).astype(jnp.float32)
    ox = jnp.sum(alpha * hx[None, :], axis=1)
    oy = jnp.sum(alpha * hy[None, :], axis=1)
    out_ref[...] = jnp.stack([ox, oy], axis=1)


def _flash_gat(adj, sums, sums_t, inv_counts, av, br):
    n = adj.shape[0]
    return pl.pallas_call(
        _flash_gat_body,
        grid=(n // br,),
        in_specs=[
            pl.BlockSpec((br, n), lambda i: (i, 0)),
            pl.BlockSpec((br, 2), lambda i: (i, 0)),
            pl.BlockSpec((2, n), lambda i: (0, 0)),
            pl.BlockSpec((br, 1), lambda i: (i, 0)),
            pl.BlockSpec((1, n), lambda i: (0, 0)),
            pl.BlockSpec((1, 4), lambda i: (0, 0)),
        ],
        out_specs=pl.BlockSpec((br, 2), lambda i: (i, 0)),
        out_shape=jax.ShapeDtypeStruct((n, 2), jnp.float32),
        interpret=_INTERPRET,
    )(adj, sums, sums_t, inv_counts.reshape(n, 1), inv_counts.reshape(1, n),
      av)


def kernel(inputs, offsets, adj_matrix, word_vectors, W, a_src, a_dst):
    n = adj_matrix.shape[0]
    t_tok = inputs.shape[0]

    hw = _vocab_matmul(word_vectors, W, bv=2000)

    seg = jnp.searchsorted(
        offsets, jnp.arange(t_tok, dtype=offsets.dtype), side="right") - 1
    g = jnp.take(hw, inputs, axis=0)
    sums = jax.ops.segment_sum(g, seg, num_segments=n)

    counts = jnp.diff(offsets, append=jnp.int32(t_tok)).astype(jnp.float32)
    inv_counts = 1.0 / jnp.maximum(counts, 1.0)

    av = jnp.concatenate([a_src, a_dst]).reshape(1, 4)
    return _flash_gat(adj_matrix, sums, sums.T, inv_counts, av, br=200)


# speedup vs baseline: 1.1257x; 1.0715x over previous
import functools

import jax
import jax.numpy as jnp
from jax import lax
from jax.experimental import pallas as pl
from jax.experimental.pallas import tpu as pltpu
from jax.experimental.pallas import tpu_sc as plsc

_INTERPRET = False


def _vocab_matmul_body(wv_ref, w_ref, out_ref):
    out_ref[...] = jnp.dot(wv_ref[...], w_ref[...],
                           preferred_element_type=jnp.float32,
                           precision=jax.lax.Precision.HIGHEST)


def _vocab_matmul(wv, W, bv):
    V, D = wv.shape
    out_dim = W.shape[1]
    return pl.pallas_call(
        _vocab_matmul_body,
        grid=(V // bv,),
        in_specs=[
            pl.BlockSpec((bv, D), lambda i: (i, 0)),
            pl.BlockSpec((D, out_dim), lambda i: (0, 0)),
        ],
        out_specs=pl.BlockSpec((bv, out_dim), lambda i: (i, 0)),
        out_shape=jax.ShapeDtypeStruct((V, out_dim), jnp.float32),
        interpret=_INTERPRET,
    )(wv, W)


_SC_CHUNK = 128


def _sc_bag(hw, tok_pad, seg_pad, zeros, n_acc, n_steps):
    info = plsc.get_sparse_core_info()
    nc, ns = info.num_cores, info.num_subcores
    mesh = plsc.VectorSubcoreMesh(core_axis_name="c", subcore_axis_name="s")

    @functools.partial(
        pl.kernel, mesh=mesh,
        out_type=jax.ShapeDtypeStruct((nc, n_acc, 8), jnp.float32),
        scratch_types=[
            pltpu.VMEM((n_steps, _SC_CHUNK), jnp.int32),
            pltpu.VMEM((n_steps, _SC_CHUNK), jnp.int32),
            pltpu.VMEM((_SC_CHUNK, 8), jnp.float32),
            pltpu.VMEM_SHARED((n_acc, 8), jnp.float32),
            pltpu.SemaphoreType.DMA,
        ],
        compiler_params=pltpu.CompilerParams(use_tc_tiling_on_sc=False),
    )
    def bag(hw_hbm, tok_hbm, seg_hbm, zeros_hbm, out_hbm,
            tok_v, seg_v, rows_v, acc_sh, sem):
        cid = lax.axis_index("c")
        sid = lax.axis_index("s")
        wid = sid * nc + cid
        pltpu.sync_copy(tok_hbm.at[wid], tok_v)
        pltpu.sync_copy(seg_hbm.at[wid], seg_v)

        @pl.when(sid == 0)
        def _():
            pltpu.sync_copy(zeros_hbm, acc_sh)

        plsc.subcore_barrier()

        def body(j, carry):
            pltpu.async_copy(hw_hbm.at[tok_v.at[j]], rows_v, sem).wait()
            pltpu.sync_copy(rows_v, acc_sh.at[seg_v.at[j]], add=True)
            return carry

        lax.fori_loop(0, n_steps, body, 0)
        plsc.subcore_barrier()

        @pl.when(sid == 0)
        def _():
            pltpu.sync_copy(acc_sh, out_hbm.at[cid])

    return bag(hw, tok_pad, seg_pad, zeros)


def _flash_gat_body(adj_ref, sums_r_ref, ic_r_ref, sums_t_ref, ic_t_ref,
                    sums_f_ref, ic_f_ref, av_ref, out_ref):
    n = adj_ref.shape[1]
    hr = sums_r_ref[...] * ic_r_ref[...]
    ht = sums_t_ref[...] * ic_t_ref[...]
    h16 = (sums_f_ref[...] * ic_f_ref[...]).astype(jnp.bfloat16)
    a00 = av_ref[0, 0]
    a01 = av_ref[0, 1]
    a10 = av_ref[0, 2]
    a11 = av_ref[0, 3]
    s = hr[:, 0] * a00 + hr[:, 1] * a01
    t = ht[0, :] * a10 + ht[1, :] * a11
    es1 = jnp.exp(s)[:, None]
    es2 = jnp.exp(0.2 * s)[:, None]
    et1 = jnp.exp(t)[None, :]
    et2 = jnp.exp(0.2 * t)[None, :]
    x = s[:, None] + t[None, :]
    pe = jnp.where(x >= 0, es1 * et1, es2 * et2)
    pm = jnp.where(adj_ref[...] > 0.5, pe, jnp.float32(0.0))
    denom = jnp.sum(pm, axis=1, keepdims=True)
    full = denom == 0.0
    alpha = (jnp.where(full, jnp.float32(1.0), pm)
             / jnp.where(full, jnp.float32(n), denom))
    out_ref[...] = jnp.dot(alpha.astype(jnp.bfloat16), h16,
                           preferred_element_type=jnp.float32)


def _flash_gat(adj, sums, sums_t, inv_counts, av, br):
    n = adj.shape[0]
    return pl.pallas_call(
        _flash_gat_body,
        grid=(n // br,),
        in_specs=[
            pl.BlockSpec((br, n), lambda i: (i, 0)),
            pl.BlockSpec((br, 2), lambda i: (i, 0)),
            pl.BlockSpec((br, 1), lambda i: (i, 0)),
            pl.BlockSpec((2, n), lambda i: (0, 0)),
            pl.BlockSpec((1, n), lambda i: (0, 0)),
            pl.BlockSpec((n, 2), lambda i: (0, 0)),
            pl.BlockSpec((n, 1), lambda i: (0, 0)),
            pl.BlockSpec((1, 4), lambda i: (0, 0)),
        ],
        out_specs=pl.BlockSpec((br, 2), lambda i: (i, 0)),
        out_shape=jax.ShapeDtypeStruct((n, 2), jnp.float32),
        interpret=_INTERPRET,
    )(adj, sums, inv_counts.reshape(n, 1), sums_t, inv_counts.reshape(1, n),
      sums, inv_counts.reshape(n, 1), av)


def kernel(inputs, offsets, adj_matrix, word_vectors, W, a_src, a_dst):
    n = adj_matrix.shape[0]
    t_tok = inputs.shape[0]
    n_workers = 32
    n_steps = -(-t_tok // (n_workers * _SC_CHUNK))
    t_pad = n_workers * n_steps * _SC_CHUNK
    n_acc = n + 8

    w_pad = jnp.pad(W, ((0, 0), (0, 8 - W.shape[1])))
    hw = _vocab_matmul(word_vectors, w_pad, bv=2000)

    seg = jnp.searchsorted(
        offsets, jnp.arange(t_tok, dtype=offsets.dtype), side="right") - 1
    tok_pad = jnp.pad(inputs, (0, t_pad - t_tok)).reshape(
        n_workers, n_steps, _SC_CHUNK)
    seg_pad = jnp.pad(seg, (0, t_pad - t_tok),
                      constant_values=n).reshape(n_workers, n_steps, _SC_CHUNK)
    zeros = jnp.zeros((n_acc, 8), jnp.float32)
    parts = _sc_bag(hw, tok_pad, seg_pad, zeros, n_acc, n_steps)
    sums = (parts[0, :n, :2] + parts[1, :n, :2])

    counts = jnp.diff(offsets, append=jnp.int32(t_tok)).astype(jnp.float32)
    inv_counts = 1.0 / jnp.maximum(counts, 1.0)

    av = jnp.concatenate([a_src, a_dst]).reshape(1, 4)
    return _flash_gat(adj_matrix, sums, sums.T, inv_counts, av, br=200)


# speedup vs baseline: 26.0361x; 23.1296x over previous
import functools

import jax
import jax.numpy as jnp
from jax import lax
from jax.experimental import pallas as pl
from jax.experimental.pallas import tpu as pltpu
from jax.experimental.pallas import tpu_sc as plsc

_INTERPRET = False


def _vocab_matmul_body(wv_ref, w_ref, out_ref):
    out_ref[...] = jnp.dot(wv_ref[...], w_ref[...],
                           preferred_element_type=jnp.float32,
                           precision=jax.lax.Precision.HIGHEST)


def _vocab_matmul(wv, W, bv):
    V, D = wv.shape
    out_dim = W.shape[1]
    return pl.pallas_call(
        _vocab_matmul_body,
        grid=(V // bv,),
        in_specs=[
            pl.BlockSpec((bv, D), lambda i: (i, 0)),
            pl.BlockSpec((D, out_dim), lambda i: (0, 0)),
        ],
        out_specs=pl.BlockSpec((bv, out_dim), lambda i: (i, 0)),
        out_shape=jax.ShapeDtypeStruct((V, out_dim), jnp.float32),
        interpret=_INTERPRET,
    )(wv, W)


_SC_CHUNK = 128


def _sc_bag(hw, tok_pad, seg_pad, zeros, n_acc, n_steps):
    info = plsc.get_sparse_core_info()
    nc, ns = info.num_cores, info.num_subcores
    mesh = plsc.VectorSubcoreMesh(core_axis_name="c", subcore_axis_name="s")

    @functools.partial(
        pl.kernel, mesh=mesh,
        out_type=jax.ShapeDtypeStruct((nc, n_acc, 8), jnp.float32),
        scratch_types=[
            pltpu.VMEM((n_steps, _SC_CHUNK), jnp.int32),
            pltpu.VMEM((n_steps, _SC_CHUNK), jnp.int32),
            pltpu.VMEM((_SC_CHUNK, 8), jnp.float32),
            pltpu.VMEM_SHARED((n_acc, 8), jnp.float32),
            pltpu.SemaphoreType.DMA,
        ],
        compiler_params=pltpu.CompilerParams(use_tc_tiling_on_sc=False),
    )
    def bag(hw_hbm, tok_hbm, seg_hbm, zeros_hbm, out_hbm,
            tok_v, seg_v, rows_v, acc_sh, sem):
        cid = lax.axis_index("c")
        sid = lax.axis_index("s")
        wid = sid * nc + cid
        pltpu.sync_copy(tok_hbm.at[wid], tok_v)
        pltpu.sync_copy(seg_hbm.at[wid], seg_v)

        @pl.when(sid == 0)
        def _():
            pltpu.sync_copy(zeros_hbm, acc_sh)

        plsc.subcore_barrier()

        def body(j, carry):
            pltpu.async_copy(hw_hbm.at[tok_v.at[j]], rows_v, sem).wait()
            pltpu.sync_copy(rows_v, acc_sh.at[seg_v.at[j]], add=True)
            return carry

        lax.fori_loop(0, n_steps, body, 0)
        plsc.subcore_barrier()

        @pl.when(sid == 0)
        def _():
            pltpu.sync_copy(acc_sh, out_hbm.at[cid])

    return bag(hw, tok_pad, seg_pad, zeros)


def _flash_gat_body(adj_ref, sums_r_ref, ic_r_ref, sums_t_ref, ic_t_ref,
                    sums_f_ref, ic_f_ref, av_ref, out_ref):
    n = adj_ref.shape[1]
    hr = sums_r_ref[...] * ic_r_ref[...]
    ht = sums_t_ref[...] * ic_t_ref[...]
    h16 = (sums_f_ref[...] * ic_f_ref[...]).astype(jnp.bfloat16)
    a00 = av_ref[0, 0]
    a01 = av_ref[0, 1]
    a10 = av_ref[0, 2]
    a11 = av_ref[0, 3]
    s = hr[:, 0] * a00 + hr[:, 1] * a01
    t = ht[0, :] * a10 + ht[1, :] * a11
    es1 = jnp.exp(s)[:, None]
    es2 = jnp.exp(0.2 * s)[:, None]
    et1 = jnp.exp(t)[None, :]
    et2 = jnp.exp(0.2 * t)[None, :]
    x = s[:, None] + t[None, :]
    pe = jnp.where(x >= 0, es1 * et1, es2 * et2)
    pm = jnp.where(adj_ref[...] > 0.5, pe, jnp.float32(0.0))
    denom = jnp.sum(pm, axis=1, keepdims=True)
    full = denom == 0.0
    alpha = (jnp.where(full, jnp.float32(1.0), pm)
             / jnp.where(full, jnp.float32(n), denom))
    out_ref[...] = jnp.dot(alpha.astype(jnp.bfloat16), h16,
                           preferred_element_type=jnp.float32)


def _flash_gat(adj, sums, sums_t, inv_counts, av, br):
    n = adj.shape[0]
    return pl.pallas_call(
        _flash_gat_body,
        grid=(n // br,),
        in_specs=[
            pl.BlockSpec((br, n), lambda i: (i, 0)),
            pl.BlockSpec((br, 2), lambda i: (i, 0)),
            pl.BlockSpec((br, 1), lambda i: (i, 0)),
            pl.BlockSpec((2, n), lambda i: (0, 0)),
            pl.BlockSpec((1, n), lambda i: (0, 0)),
            pl.BlockSpec((n, 2), lambda i: (0, 0)),
            pl.BlockSpec((n, 1), lambda i: (0, 0)),
            pl.BlockSpec((1, 4), lambda i: (0, 0)),
        ],
        out_specs=pl.BlockSpec((br, 2), lambda i: (i, 0)),
        out_shape=jax.ShapeDtypeStruct((n, 2), jnp.float32),
        interpret=_INTERPRET,
    )(adj, sums, inv_counts.reshape(n, 1), sums_t, inv_counts.reshape(1, n),
      sums, inv_counts.reshape(n, 1), av)


def kernel(inputs, offsets, adj_matrix, word_vectors, W, a_src, a_dst):
    n = adj_matrix.shape[0]
    t_tok = inputs.shape[0]
    n_workers = 32
    n_steps = -(-t_tok // (n_workers * _SC_CHUNK))
    t_pad = n_workers * n_steps * _SC_CHUNK
    n_acc = n + 8

    w_pad = jnp.pad(W, ((0, 0), (0, 8 - W.shape[1])))
    hw = _vocab_matmul(word_vectors, w_pad, bv=2000)

    bc = jnp.zeros((t_tok,), jnp.int32).at[offsets].add(1)
    seg = jnp.cumsum(bc) - 1
    tok_pad = jnp.pad(inputs, (0, t_pad - t_tok)).reshape(
        n_workers, n_steps, _SC_CHUNK)
    seg_pad = jnp.pad(seg, (0, t_pad - t_tok),
                      constant_values=n).reshape(n_workers, n_steps, _SC_CHUNK)
    zeros = jnp.zeros((n_acc, 8), jnp.float32)
    parts = _sc_bag(hw, tok_pad, seg_pad, zeros, n_acc, n_steps)
    sums = (parts[0, :n, :2] + parts[1, :n, :2])

    counts = jnp.diff(offsets, append=jnp.int32(t_tok)).astype(jnp.float32)
    inv_counts = 1.0 / jnp.maximum(counts, 1.0)

    av = jnp.concatenate([a_src, a_dst]).reshape(1, 4)
    return _flash_gat(adj_matrix, sums, sums.T, inv_counts, av, br=200)
